# Initial kernel scaffold; baseline (speedup 1.0000x reference)
import jax
import jax.numpy as jnp
from jax.experimental import pallas as pl


def kernel(x, edge_index, edge_attribute, W_rel1, W_root1, b1, W_rel2, W_root2, b2, Wl1, bl1, Wl2, bl2, Wlast, blast):
    raise NotImplementedError("write your pallas kernel here")


# speedup vs baseline: 5.4680x; 5.4680x over previous
import functools

import jax
import jax.numpy as jnp
from jax import lax
from jax.experimental import pallas as pl
from jax.experimental.pallas import tpu as pltpu
from jax.experimental.pallas import tpu_sc as plsc

N_NODES = 10000
FDIM = 128
E_TOTAL = 320000
NC = 2
NS = 16
NW = NC * NS
CHUNK = 128
NCHUNKS = E_TOTAL // CHUNK
ROWS_PER_TILE = 624
ROWS_REM = N_NODES - NS * ROWS_PER_TILE


def _sc_agg_body(feat_hbm, src_hbm, dst_hbm, attr_hbm, zeros_hbm, out_hbm,
                 src_v, dst_v, attr_v, rows_v, acc, sem):
    c = lax.axis_index("c")
    s = lax.axis_index("s")
    wid = s * NC + c

    pltpu.sync_copy(zeros_hbm.at[pl.ds(s * ROWS_PER_TILE, ROWS_PER_TILE)],
                    acc.at[pl.ds(s * ROWS_PER_TILE, ROWS_PER_TILE)])

    @pl.when(s == NS - 1)
    def _():
        pltpu.sync_copy(zeros_hbm.at[pl.ds(NS * ROWS_PER_TILE, ROWS_REM)],
                        acc.at[pl.ds(NS * ROWS_PER_TILE, ROWS_REM)])

    plsc.subcore_barrier()

    nchunks_w = (NCHUNKS - wid + NW - 1) // NW

    def chunk_body(i, carry):
        ch = wid + i * NW
        base = ch * CHUNK
        pltpu.sync_copy(src_hbm.at[pl.ds(base, CHUNK)], src_v)
        pltpu.sync_copy(dst_hbm.at[pl.ds(base, CHUNK)], dst_v)
        pltpu.sync_copy(attr_hbm.at[pl.ds(base, CHUNK)], attr_v)
        pltpu.async_copy(feat_hbm.at[src_v], rows_v, sem).wait()

        def scale_body(g, carry2):
            a16 = attr_v[pl.ds(g * 16, 16)]
            for j in range(16):
                e = g * 16 + j
                a = jnp.full((16,), a16[j], jnp.float32)
                for k in range(8):
                    sl = pl.ds(k * 16, 16)
                    rows_v[e, sl] = rows_v[e, sl] * a
            return carry2

        lax.fori_loop(0, CHUNK // 16, scale_body, 0, unroll=False)
        pltpu.sync_copy(rows_v, acc.at[dst_v], add=True)
        return carry

    lax.fori_loop(0, nchunks_w, chunk_body, 0, unroll=False)

    plsc.subcore_barrier()
    pltpu.sync_copy(acc.at[pl.ds(s * ROWS_PER_TILE, ROWS_PER_TILE)],
                    out_hbm.at[c, pl.ds(s * ROWS_PER_TILE, ROWS_PER_TILE)])

    @pl.when(s == NS - 1)
    def _():
        pltpu.sync_copy(acc.at[pl.ds(NS * ROWS_PER_TILE, ROWS_REM)],
                        out_hbm.at[c, pl.ds(NS * ROWS_PER_TILE, ROWS_REM)])


@jax.jit
def _sc_edge_agg(feat, src, dst, attr, zeros):
    mesh = plsc.VectorSubcoreMesh(core_axis_name="c", subcore_axis_name="s")
    kern = pl.kernel(
        _sc_agg_body,
        mesh=mesh,
        out_type=jax.ShapeDtypeStruct((NC, N_NODES, FDIM), jnp.float32),
        scratch_types=[
            pltpu.VMEM((CHUNK,), jnp.int32),
            pltpu.VMEM((CHUNK,), jnp.int32),
            pltpu.VMEM((CHUNK,), jnp.float32),
            pltpu.VMEM((CHUNK, FDIM), jnp.float32),
            pltpu.VMEM_SHARED((N_NODES, FDIM), jnp.float32),
            pltpu.SemaphoreType.DMA,
        ],
    )
    return kern(feat, src, dst, attr, zeros)


def _dense1_body(agg0_ref, agg1_ref, x_ref, wr_ref, wrt_ref, b_ref, wr2_ref,
                 h1_ref, t_ref):
    agg = agg0_ref[...] + agg1_ref[...]
    h1 = jnp.dot(agg, wr_ref[...], preferred_element_type=jnp.float32)
    h1 += jnp.dot(x_ref[...], wrt_ref[...], preferred_element_type=jnp.float32)
    h1 = jnp.maximum(h1 + b_ref[...], 0.0)
    h1_ref[...] = h1
    t_ref[...] = jnp.dot(h1, wr2_ref[...], preferred_element_type=jnp.float32)


@jax.jit
def _dense1(agg0, agg1, x, W_rel1, W_root1, b1, W_rel2):
    R = 1000
    grid = N_NODES // R
    full = lambda shape: pl.BlockSpec(shape, lambda i: (0, 0))
    return pl.pallas_call(
        _dense1_body,
        grid=(grid,),
        in_specs=[
            pl.BlockSpec((R, FDIM), lambda i: (i, 0)),
            pl.BlockSpec((R, FDIM), lambda i: (i, 0)),
            pl.BlockSpec((R, FDIM), lambda i: (i, 0)),
            full((FDIM, 256)),
            full((FDIM, 256)),
            full((1, 256)),
            full((256, FDIM)),
        ],
        out_specs=[
            pl.BlockSpec((R, 256), lambda i: (i, 0)),
            pl.BlockSpec((R, FDIM), lambda i: (i, 0)),
        ],
        out_shape=[
            jax.ShapeDtypeStruct((N_NODES, 256), jnp.float32),
            jax.ShapeDtypeStruct((N_NODES, FDIM), jnp.float32),
        ],
        compiler_params=pltpu.CompilerParams(
            dimension_semantics=("parallel",)),
    )(agg0, agg1, x, W_rel1, W_root1, b1, W_rel2)


def _dense2_body(agg0_ref, agg1_ref, h1_ref, wrt2_ref, b2_ref, wl1_ref,
                 bl1_ref, wl2_ref, bl2_ref, h4_ref):
    h2 = agg0_ref[...] + agg1_ref[...]
    h2 += jnp.dot(h1_ref[...], wrt2_ref[...], preferred_element_type=jnp.float32)
    h2 = jnp.maximum(h2 + b2_ref[...], 0.0)
    h3 = jnp.dot(h2, wl1_ref[...], preferred_element_type=jnp.float32)
    h3 = jnp.maximum(h3 + bl1_ref[...], 0.0)
    h4 = jnp.dot(h3, wl2_ref[...], preferred_element_type=jnp.float32)
    h4_ref[...] = h4 + bl2_ref[...]


@jax.jit
def _dense2(agg0, agg1, h1, W_root2, b2, Wl1, bl1, Wl2, bl2):
    R = 1000
    grid = N_NODES // R
    full = lambda shape: pl.BlockSpec(shape, lambda i: (0, 0))
    return pl.pallas_call(
        _dense2_body,
        grid=(grid,),
        in_specs=[
            pl.BlockSpec((R, FDIM), lambda i: (i, 0)),
            pl.BlockSpec((R, FDIM), lambda i: (i, 0)),
            pl.BlockSpec((R, 256), lambda i: (i, 0)),
            full((256, FDIM)),
            full((1, FDIM)),
            full((FDIM, 64)),
            full((1, 64)),
            full((64, 1)),
            full((1, 1)),
        ],
        out_specs=pl.BlockSpec((R, 1), lambda i: (i, 0)),
        out_shape=jax.ShapeDtypeStruct((N_NODES, 1), jnp.float32),
        compiler_params=pltpu.CompilerParams(
            dimension_semantics=("parallel",)),
    )(agg0, agg1, h1, W_root2, b2, Wl1, bl1, Wl2, bl2)


def _final_body(h_ref, wlast_ref, blast_ref, out_ref):
    o = jnp.dot(h_ref[...], wlast_ref[...], preferred_element_type=jnp.float32)
    out_ref[...] = jax.nn.sigmoid(o + blast_ref[...])


@jax.jit
def _final(H, Wlast, blast):
    return pl.pallas_call(
        _final_body,
        out_shape=jax.ShapeDtypeStruct((100, 1), jnp.float32),
    )(H, Wlast, blast)


def kernel(x, edge_index, edge_attribute, W_rel1, W_root1, b1, W_rel2,
           W_root2, b2, Wl1, bl1, Wl2, bl2, Wlast, blast):
    src = edge_index[0]
    dst = edge_index[1]
    zeros = jnp.zeros((N_NODES, FDIM), jnp.float32)

    aggp1 = _sc_edge_agg(x, src, dst, edge_attribute, zeros)
    h1, t = _dense1(aggp1[0], aggp1[1], x, W_rel1, W_root1,
                    b1.reshape(1, 256), W_rel2)
    aggp2 = _sc_edge_agg(t, src, dst, edge_attribute, zeros)
    h4 = _dense2(aggp2[0], aggp2[1], h1, W_root2, b2.reshape(1, FDIM),
                 Wl1, bl1.reshape(1, 64), Wl2, bl2.reshape(1, 1))
    H = h4.reshape(100, 100)
    return _final(H, Wlast, blast.reshape(1, 1))
